# baseline (speedup 1.0000x reference)
import functools

import jax
import jax.numpy as jnp
from jax import lax
from jax.experimental import pallas as pl
from jax.experimental.pallas import tpu as pltpu
from jax.experimental.pallas import tpu_sc as plsc

H = 1024
FF = 4096
E = 8
TOPK = 2
TM = 256
TK = 512
KB = FF // TK
T = 4096
MT = T // TM + E
APAD = MT * TM


def _sc_gather_rows(table, idx):
    B = idx.shape[0]
    D = table.shape[1]
    info = plsc.get_sparse_core_info()
    nc, ns = info.num_cores, info.num_subcores
    nw = nc * ns
    b_per_w = B // nw
    ch = 64 if b_per_w % 64 == 0 else b_per_w
    nch = b_per_w // ch
    mesh = plsc.VectorSubcoreMesh(core_axis_name="c", subcore_axis_name="s")

    @functools.partial(
        pl.kernel, mesh=mesh,
        out_type=jax.ShapeDtypeStruct((B, D), table.dtype),
        scratch_types=[
            pltpu.VMEM((ch,), jnp.int32),
            pltpu.VMEM((ch, D), table.dtype),
            pltpu.SemaphoreType.DMA,
        ],
    )
    def k(table_hbm, idx_hbm, out_hbm, idx_v, rows_v, sem):
        wid = lax.axis_index("s") * nc + lax.axis_index("c")
        base = wid * b_per_w

        def body(i, carry):
            off = base + i * ch
            pltpu.sync_copy(idx_hbm.at[pl.ds(off, ch)], idx_v)
            pltpu.async_copy(table_hbm.at[idx_v], rows_v, sem).wait()
            pltpu.sync_copy(rows_v, out_hbm.at[pl.ds(off, ch)])
            return carry

        lax.fori_loop(0, nch, body, 0)

    return k(table, idx)


def _moe_body(g_ref, x_ref, w1_ref, b1_ref, w2_ref, b2_ref, o_ref):
    p = pl.program_id(1)
    k = pl.program_id(2)

    @pl.when((p == 0) & (k == 0))
    def _():
        o_ref[...] = jnp.zeros_like(o_ref)

    @pl.when(k == 0)
    def _():
        o_ref[...] += (0.5 * b2_ref[0, 0])[None, :]

    xb = x_ref[...].astype(jnp.bfloat16)
    h = jnp.dot(xb, w1_ref[0], preferred_element_type=jnp.float32)
    h = jax.nn.relu(h + b1_ref[0, 0, 0][None, :]) * 0.5
    o_ref[...] += jnp.dot(h.astype(jnp.bfloat16), w2_ref[0],
                          preferred_element_type=jnp.float32)


def _moe_tc(g_tile, x_s, w1b, b1r, w2b, b2r):
    grid = (MT, 2, KB)

    def e_idx(m, p, k, g):
        return lax.rem(g[m] + p, E)

    grid_spec = pltpu.PrefetchScalarGridSpec(
        num_scalar_prefetch=1,
        grid=grid,
        in_specs=[
            pl.BlockSpec((TM, H), lambda m, p, k, g: (m, 0)),
            pl.BlockSpec((1, H, TK), lambda m, p, k, g: (e_idx(m, p, k, g), 0, k)),
            pl.BlockSpec((1, 1, 1, TK), lambda m, p, k, g: (e_idx(m, p, k, g), k, 0, 0)),
            pl.BlockSpec((1, TK, H), lambda m, p, k, g: (e_idx(m, p, k, g), k, 0)),
            pl.BlockSpec((1, 1, H), lambda m, p, k, g: (e_idx(m, p, k, g), 0, 0)),
        ],
        out_specs=pl.BlockSpec((TM, H), lambda m, p, k, g: (m, 0)),
    )
    return pl.pallas_call(
        _moe_body,
        grid_spec=grid_spec,
        out_shape=jax.ShapeDtypeStruct((APAD, H), jnp.float32),
        compiler_params=pltpu.CompilerParams(
            dimension_semantics=("parallel", "arbitrary", "arbitrary")),
    )(g_tile, x_s, w1b, b1r, w2b, b2r)


def kernel(x, W1, b1, W2, b2):
    B, S, _ = x.shape
    x_flat = x.reshape(-1, H)

    hv = x_flat[:, :TOPK].sum(axis=1).astype(jnp.int32) % E
    counts = jnp.bincount(hv, length=E)
    tiles_g = (counts + TM - 1) // TM
    cum_tiles = jnp.cumsum(tiles_g)
    pad_off = (cum_tiles - tiles_g) * TM
    oh = (hv[:, None] == jnp.arange(E)[None, :]).astype(jnp.int32)
    rank = jnp.take_along_axis(jnp.cumsum(oh, axis=0), hv[:, None], axis=1)[:, 0] - 1
    pos = (pad_off[hv] + rank).astype(jnp.int32)
    src_rows = jnp.zeros((APAD,), jnp.int32).at[pos].set(
        jnp.arange(T, dtype=jnp.int32))
    g_tile = jnp.minimum(
        jnp.searchsorted(cum_tiles, jnp.arange(MT), side="right"), E - 1
    ).astype(jnp.int32)

    x_s = _sc_gather_rows(x_flat, src_rows)

    w1b = W1.astype(jnp.bfloat16)
    w2b = W2.astype(jnp.bfloat16)
    b1r = b1.reshape(E, KB, 1, TK)
    b2r = b2.reshape(E, 1, H)
    y_s = _moe_tc(g_tile, x_s, w1b, b1r, w2b, b2r)

    out_flat = _sc_gather_rows(y_s, pos)
    return out_flat.reshape(B, S, H)
